# baseline (speedup 1.0000x reference)
import jax
import jax.numpy as jnp
from jax.experimental import pallas as pl

_BLOCK_ROWS = 8000


def _copy_scatter_body(x_ref, o_ref):
    i = pl.program_id(0)

    @pl.when(i == 0)
    def _patch_block():
        blk = x_ref[...]
        r = jax.lax.broadcasted_iota(jnp.int32, blk.shape, 0)
        c = jax.lax.broadcasted_iota(jnp.int32, blk.shape, 1)
        col0 = c == 0
        blk = jnp.where((r == 0) & col0, jnp.float32(88.0), blk)
        blk = jnp.where((r == 1) & col0, jnp.float32(99.0), blk)
        o_ref[...] = blk

    @pl.when(i > 0)
    def _copy_block():
        o_ref[...] = x_ref[...]


def kernel(x):
    n, d = x.shape
    grid = pl.cdiv(n, _BLOCK_ROWS)
    return pl.pallas_call(
        _copy_scatter_body,
        grid=(grid,),
        in_specs=[pl.BlockSpec((_BLOCK_ROWS, d), lambda i: (i, 0))],
        out_specs=pl.BlockSpec((_BLOCK_ROWS, d), lambda i: (i, 0)),
        out_shape=jax.ShapeDtypeStruct((n, d), x.dtype),
    )(x)
